# baseline (speedup 1.0000x reference)
import functools
import jax
import jax.numpy as jnp
from jax import lax
from jax.experimental import pallas as pl
from jax.experimental.pallas import tpu as pltpu
from jax.experimental.pallas import tpu_sc as plsc


def _make_sc_kernel(b, h, w, d):
    mesh = plsc.VectorSubcoreMesh(core_axis_name="c", subcore_axis_name="s")
    info = plsc.get_sparse_core_info()
    nc = info.num_cores

    @functools.partial(
        pl.kernel,
        mesh=mesh,
        out_type=jax.ShapeDtypeStruct((b, h, w, 2 * d), jnp.float32),
        scratch_types=[
            pltpu.VMEM((w, 2 * d), jnp.float32),
            pltpu.SemaphoreType.DMA,
        ],
    )
    def k(col_hbm, row_hbm, out_hbm, slice_v, sem):
        wid = lax.axis_index("s") * nc + lax.axis_index("c")
        for xx in range(w):
            pltpu.sync_copy(col_hbm.at[xx], slice_v.at[xx, pl.ds(0, d)])
            pltpu.sync_copy(row_hbm.at[wid], slice_v.at[xx, pl.ds(d, d)])
        copies = [
            pltpu.make_async_copy(slice_v, out_hbm.at[bb, wid], sem)
            for bb in range(b)
        ]
        for c in copies:
            c.start()
        for c in copies:
            c.wait()

    return k


def kernel(x, row_emb, col_emb):
    b = x.shape[0]
    h, w = x.shape[-2], x.shape[-1]
    d = row_emb.shape[1]
    k = _make_sc_kernel(b, h, w, d)
    out = k(col_emb, row_emb)
    return jnp.transpose(out, (0, 3, 1, 2))


# speedup vs baseline: 1.6636x; 1.6636x over previous
import functools
import jax
import jax.numpy as jnp
from jax import lax
from jax.experimental import pallas as pl
from jax.experimental.pallas import tpu as pltpu
from jax.experimental.pallas import tpu_sc as plsc


def _make_sc_kernel(b, h, w, d):
    mesh = plsc.VectorSubcoreMesh(core_axis_name="c", subcore_axis_name="s")
    info = plsc.get_sparse_core_info()
    nc = info.num_cores

    @functools.partial(
        pl.kernel,
        mesh=mesh,
        out_type=jax.ShapeDtypeStruct((b, h, w, 2 * d), jnp.float32),
        scratch_types=[
            pltpu.VMEM((w, 2 * d), jnp.float32),
            pltpu.SemaphoreType.DMA,
        ],
    )
    def k(col_hbm, row_hbm, out_hbm, slice_v, sem):
        wid = lax.axis_index("s") * nc + lax.axis_index("c")
        fills = []
        for xx in range(w):
            fills.append(pltpu.make_async_copy(
                col_hbm.at[xx], slice_v.at[xx, pl.ds(0, d)], sem))
            fills.append(pltpu.make_async_copy(
                row_hbm.at[wid], slice_v.at[xx, pl.ds(d, d)], sem))
        for f in fills:
            f.start()
        for f in fills:
            f.wait()
        copies = [
            pltpu.make_async_copy(slice_v, out_hbm.at[bb, wid], sem)
            for bb in range(b)
        ]
        for c in copies:
            c.start()
        for c in copies:
            c.wait()

    return k


def kernel(x, row_emb, col_emb):
    b = x.shape[0]
    h, w = x.shape[-2], x.shape[-1]
    d = row_emb.shape[1]
    k = _make_sc_kernel(b, h, w, d)
    out = k(col_emb, row_emb)
    return jnp.transpose(out, (0, 3, 1, 2))
